# baseline (speedup 1.0000x reference)
import functools

import jax
import jax.numpy as jnp
from jax import lax
from jax.experimental import pallas as pl
from jax.experimental.pallas import tpu as pltpu
from jax.experimental.pallas import tpu_sc as plsc

KNN = 20
KPAD = 24


def _bn_scale_shift(p):
    s = p['g'] / jnp.sqrt(p['v'] + 1e-5)
    t = p['b'] - s * p['m']
    return s, t


def _prep_topk_body(x_ref, wc_ref, wd_ref, t_ref, idx_ref, a_ref, b_ref, *, n, k, kpad):
    b = pl.program_id(0)
    X = x_ref[0]
    a_ref[0] = jnp.dot(X, wc_ref[...], preferred_element_type=jnp.float32) + t_ref[...]
    b_ref[0] = jnp.dot(X, wd_ref[...], preferred_element_type=jnp.float32)
    G = lax.dot_general(X, X, (((1,), (1,)), ((), ())),
                        preferred_element_type=jnp.float32)
    xx = jnp.sum(X * X, axis=1, keepdims=True)
    work = xx + jnp.reshape(xx, (1, n)) - 2.0 * G
    iota = lax.broadcasted_iota(jnp.int32, (n, n), 1)
    base = b * n
    am = None
    for t in range(k):
        rowmin = jnp.min(work, axis=1, keepdims=True)
        cand = jnp.where(work == rowmin, iota, n)
        am = jnp.min(cand, axis=1)
        idx_ref[0, :, t] = am + base
        work = jnp.where(iota == am[:, None], jnp.inf, work)
    for t in range(k, kpad):
        idx_ref[0, :, t] = am + base


def _layer_tc(xT, wc_t, wd_t, tvec):
    B, N, Cp = xT.shape
    O = wc_t.shape[1]
    idx, Ap, Bp = pl.pallas_call(
        functools.partial(_prep_topk_body, n=N, k=KNN, kpad=KPAD),
        grid=(B,),
        in_specs=[
            pl.BlockSpec((1, N, Cp), lambda b: (b, 0, 0)),
            pl.BlockSpec((Cp, O), lambda b: (0, 0)),
            pl.BlockSpec((Cp, O), lambda b: (0, 0)),
            pl.BlockSpec((1, O), lambda b: (0, 0)),
        ],
        out_specs=[
            pl.BlockSpec((1, N, KPAD), lambda b: (b, 0, 0)),
            pl.BlockSpec((1, N, O), lambda b: (b, 0, 0)),
            pl.BlockSpec((1, N, O), lambda b: (b, 0, 0)),
        ],
        out_shape=[
            jax.ShapeDtypeStruct((B, N, KPAD), jnp.int32),
            jax.ShapeDtypeStruct((B, N, O), jnp.float32),
            jax.ShapeDtypeStruct((B, N, O), jnp.float32),
        ],
    )(xT, wc_t, wd_t, tvec[None, :])
    return idx, Ap, Bp


def _sc_gather_max(idx_flat, Ap, Bp):
    BN, O = Ap.shape
    WG = Bp.shape[1]
    K = KPAD
    CH = 4
    info = plsc.get_sparse_core_info()
    NC, NS = info.num_cores, info.num_subcores
    NW = NC * NS
    rows_per = BN // NW
    nch = rows_per // CH

    mesh = plsc.VectorSubcoreMesh(core_axis_name="c", subcore_axis_name="s")

    @functools.partial(
        pl.kernel, mesh=mesh,
        out_type=jax.ShapeDtypeStruct((BN, O), jnp.float32),
        scratch_types=[
            pltpu.VMEM((CH * K,), jnp.int32),
            pltpu.VMEM((CH * K,), jnp.int32),
            pltpu.VMEM((CH * K, WG), jnp.float32),
            pltpu.VMEM((CH * K, WG), jnp.float32),
            pltpu.VMEM((CH, O), jnp.float32),
            pltpu.VMEM((CH, O), jnp.float32),
            pltpu.VMEM((CH, O), jnp.float32),
            pltpu.SemaphoreType.DMA,
            pltpu.SemaphoreType.DMA,
        ],
    )
    def sck(idx_hbm, ap_hbm, bp_hbm, out_hbm, ib0, ib1, rb0, rb1, av0, av1, ob, s0, s1):
        wid = lax.axis_index("s") * NC + lax.axis_index("c")
        base = wid * rows_per

        def issue(ch, ib, rb, av, sem):
            r0 = base + ch * CH
            pltpu.sync_copy(idx_hbm.at[pl.ds(r0 * K, CH * K)], ib)
            pltpu.async_copy(bp_hbm.at[ib], rb, sem)
            pltpu.async_copy(ap_hbm.at[pl.ds(r0, CH)], av, sem)

        def wait(ib, rb, av, sem):
            pltpu.make_async_copy(bp_hbm.at[ib], rb, sem).wait()
            pltpu.make_async_copy(ap_hbm.at[pl.ds(0, CH)], av, sem).wait()

        def compute(ch, rb, av):
            r0 = base + ch * CH
            for rr in range(CH):
                for co in range(O // 16):
                    sl = pl.ds(co * 16, 16)

                    def jb(j, acc):
                        return jnp.maximum(acc, rb[rr * K + j, sl])

                    acc = lax.fori_loop(
                        0, K, jb, jnp.full((16,), -jnp.inf, jnp.float32))
                    y = av[rr, sl] + acc
                    ob[rr, sl] = jnp.maximum(y, 0.2 * y)
            pltpu.sync_copy(ob, out_hbm.at[pl.ds(r0, CH)])

        issue(0, ib0, rb0, av0, s0)

        def body(i, carry):
            ch = 2 * i
            issue(ch + 1, ib1, rb1, av1, s1)
            wait(ib0, rb0, av0, s0)
            compute(ch, rb0, av0)

            @pl.when(ch + 2 < nch)
            def _():
                issue(ch + 2, ib0, rb0, av0, s0)

            wait(ib1, rb1, av1, s1)
            compute(ch + 1, rb1, av1)
            return carry

        lax.fori_loop(0, nch // 2, body, 0)

    return sck(idx_flat, Ap, Bp)


def _prep_tau_body(x_ref, wc_ref, wd_ref, t_ref, dist_ref, a_ref, b_ref, *, n):
    X = x_ref[0]
    a_ref[0] = jnp.dot(X, wc_ref[...], preferred_element_type=jnp.float32) + t_ref[...]
    b_ref[0] = jnp.dot(X, wd_ref[...], preferred_element_type=jnp.float32)
    G = lax.dot_general(X, X, (((1,), (1,)), ((), ())),
                        preferred_element_type=jnp.float32)
    xx = jnp.sum(X * X, axis=1, keepdims=True)
    dist = xx + jnp.reshape(xx, (1, n)) - 2.0 * G
    dist_ref[0, :, :n] = dist
    w = dist
    for _ in range(5):
        h = w.shape[1] // 2
        w = jnp.minimum(w[:, :h], w[:, h:])
    io32 = lax.broadcasted_iota(jnp.int32, (n, 32), 1)
    for _ in range(12):
        rmax = jnp.max(w, axis=1, keepdims=True)
        cand = jnp.where(w == rmax, io32, -1)
        am = jnp.max(cand, axis=1, keepdims=True)
        w = jnp.where(io32 == am, -jnp.inf, w)
    tau = jnp.max(w, axis=1, keepdims=True)
    dist_ref[0, :, n:] = jnp.broadcast_to(tau, (n, 16))


def _layer_tc2(xT, wc_t, wd_t, tvec):
    B, N, Cp = xT.shape
    O = wc_t.shape[1]
    return pl.pallas_call(
        functools.partial(_prep_tau_body, n=N),
        grid=(B,),
        in_specs=[
            pl.BlockSpec((1, N, Cp), lambda b: (b, 0, 0)),
            pl.BlockSpec((Cp, O), lambda b: (0, 0)),
            pl.BlockSpec((Cp, O), lambda b: (0, 0)),
            pl.BlockSpec((1, O), lambda b: (0, 0)),
        ],
        out_specs=[
            pl.BlockSpec((1, N, N + 16), lambda b: (b, 0, 0)),
            pl.BlockSpec((1, N, O), lambda b: (b, 0, 0)),
            pl.BlockSpec((1, N, O), lambda b: (b, 0, 0)),
        ],
        out_shape=[
            jax.ShapeDtypeStruct((B, N, N + 16), jnp.float32),
            jax.ShapeDtypeStruct((B, N, O), jnp.float32),
            jax.ShapeDtypeStruct((B, N, O), jnp.float32),
        ],
    )(xT, wc_t, wd_t, tvec[None, :])


def _sc_knn_max(dist_flat, Ap, Bp, n, idxd=None):
    BN, O = Ap.shape
    WG = Bp.shape[1]
    use_idxd = idxd is not None
    K = KNN if not use_idxd else KPAD
    CV = n + 64
    info = plsc.get_sparse_core_info()
    NC, NS = info.num_cores, info.num_subcores
    NW = NC * NS
    RPW = BN // NW

    mesh = plsc.VectorSubcoreMesh(core_axis_name="c", subcore_axis_name="s")

    NR = n + 16

    @functools.partial(
        pl.kernel, mesh=mesh,
        out_type=jax.ShapeDtypeStruct((BN, O), jnp.float32),
        compiler_params=pltpu.CompilerParams(needs_layout_passes=False),
        scratch_types=[
            pltpu.VMEM((NR,), jnp.float32),
            pltpu.VMEM((NR,), jnp.float32),
            pltpu.VMEM((CV,), jnp.float32),
            pltpu.VMEM((CV,), jnp.int32),
            pltpu.VMEM((max(K, 8),), jnp.int32),
            pltpu.VMEM((max(K, 8),), jnp.int32),
            pltpu.VMEM((32 if not use_idxd else K, WG), jnp.float32),
            pltpu.VMEM((32 if not use_idxd else K, WG), jnp.float32),
            pltpu.VMEM((1, O), jnp.float32),
            pltpu.VMEM((1, O), jnp.float32),
            pltpu.VMEM((1, O), jnp.float32),
            pltpu.VMEM((1, O), jnp.float32),
            pltpu.SemaphoreType.DMA,
            pltpu.SemaphoreType.DMA,
            pltpu.SemaphoreType.DMA,
            pltpu.SemaphoreType.DMA,
            pltpu.SemaphoreType.DMA,
            pltpu.SemaphoreType.DMA,
        ],
    )
    def sck(dist_hbm, ap_hbm, bp_hbm, idxd_hbm, out_hbm,
            db0, db1, cv, cp, gi0, gi1, gb0, gb1, ab0, ab1, ob0, ob1,
            sd0, sd1, sg0, sg1, so0, so1):
        wid = lax.axis_index("s") * NC + lax.axis_index("c")
        base = wid * RPW
        sample_base = (base // n) * n
        iota16 = lax.iota(jnp.int32, 16)
        inf16 = jnp.full((16,), jnp.inf, jnp.float32)

        def m16(av, ap_, bv, bp_):
            rbv = lax.rev(bv, (0,))
            rbp = lax.rev(bp_, (0,))
            m = av <= rbv
            lv = jnp.where(m, av, rbv)
            lp = jnp.where(m, ap_, rbp)
            hv = jnp.where(m, rbv, av)
            hp = jnp.where(m, rbp, ap_)
            lv, lp = plsc.sort_key_val(lv, lp)
            hv, hp = plsc.sort_key_val(hv, hp)
            return lv, lp, hv, hp

        def merge_chunk(state, bv, bp_):
            slo_v, slo_p, shi_v, shi_p = state
            bv, bp_ = plsc.sort_key_val(bv, bp_)
            t0v, t0p, t1v, t1p = m16(shi_v, shi_p, bv, bp_)
            u0v, u0p, u1v, u1p = m16(slo_v, slo_p, t0v, t0p)
            v0v, v0p, _, _ = m16(u1v, u1p, t1v, t1p)
            return (u0v, u0p, v0v, v0p)

        def select_issue(r, db, gi, gb, ab_, sg):
            rc = jnp.minimum(r, RPW - 1)
            tau_s = db[pl.ds(n, 16)]
            cnt = jnp.zeros((16,), jnp.int32)
            for j in range(n // 16):
                v = db[pl.ds(16 * j, 16)]
                msk = v <= tau_s
                ones = jnp.where(msk, 1, 0)
                cs = plsc.cumsum(ones)
                pos = jnp.maximum(cnt + cs - 1, 0)
                plsc.store_scatter(cv, [pos], v, mask=msk)
                plsc.store_scatter(cp, [pos], iota16 + 16 * j, mask=msk)
                cnt = cnt + plsc.all_reduce_population_count(msk)
            for q in range(3):
                plsc.store_scatter(cv, [cnt + 16 * q + iota16], inf16)
            v0 = cv[pl.ds(0, 16)]
            slo_v, slo_p = plsc.sort_key_val(v0, cp[pl.ds(0, 16)])
            state = (slo_v, slo_p, inf16, iota16)
            for cidx in range(1, 4):
                bv = cv[pl.ds(16 * cidx, 16)]
                state = merge_chunk(state, bv, cp[pl.ds(16 * cidx, 16)])
            cnts = jnp.max(cnt)
            T = jnp.maximum(0, (cnts - 49) // 16)

            def tail(jj, st):
                off = 16 * (jj + 4)
                return merge_chunk(st, cv[pl.ds(off, 16)], cp[pl.ds(off, 16)])

            slo_v, slo_p, shi_v, shi_p = lax.fori_loop(0, T, tail, state)
            if use_idxd:
                pltpu.sync_copy(idxd_hbm.at[pl.ds((base + rc) * K, K)], gi)
                pltpu.async_copy(bp_hbm.at[gi], gb, sg)
            else:
                g0 = slo_p + sample_base
                g1m = jnp.where(iota16 < 4, shi_p, slo_p) + sample_base
                pltpu.async_copy(bp_hbm.at[g0], gb.at[pl.ds(0, 16)], sg)
                pltpu.async_copy(bp_hbm.at[g1m], gb.at[pl.ds(16, 16)], sg)
            pltpu.async_copy(ap_hbm.at[pl.ds(base + rc, 1)], ab_, sg)

        def wait_gather(gi, gb, ab_, sg):
            if use_idxd:
                pltpu.make_async_copy(bp_hbm.at[gi], gb, sg).wait()
            else:
                pltpu.make_async_copy(bp_hbm.at[iota16], gb.at[pl.ds(0, 16)], sg).wait()
                pltpu.make_async_copy(bp_hbm.at[iota16], gb.at[pl.ds(16, 16)], sg).wait()
            pltpu.make_async_copy(ap_hbm.at[pl.ds(0, 1)], ab_, sg).wait()

        def issue_dist(r, db, sd):
            rc = jnp.minimum(r, RPW - 1)
            pltpu.async_copy(dist_hbm.at[pl.ds((base + rc) * NR, NR)], db, sd)

        def wait_dist(db, sd):
            pltpu.make_async_copy(dist_hbm.at[pl.ds(0, NR)], db, sd).wait()

        def reduce_write(r, gb, ab_, ob_, so_):
            for co in range(O // 16):
                sl = pl.ds(co * 16, 16)
                acc = gb[0, sl]
                for j in range(1, K):
                    acc = jnp.maximum(acc, gb[j, sl])
                y = ab_[0, sl] + acc
                ob_[0, sl] = jnp.maximum(y, 0.2 * y)
            pltpu.async_copy(ob_, out_hbm.at[pl.ds(base + r, 1)], so_)

        def wait_out(ob_, so_):
            pltpu.make_async_copy(ob_, out_hbm.at[pl.ds(0, 1)], so_).wait()

        issue_dist(0, db0, sd0)
        issue_dist(1, db1, sd1)
        wait_dist(db0, sd0)
        select_issue(0, db0, gi0, gb0, ab0, sg0)
        issue_dist(2, db0, sd0)

        def body(i, c):
            r = 2 * i
            wait_dist(db1, sd1)
            select_issue(r + 1, db1, gi1, gb1, ab1, sg1)
            issue_dist(r + 3, db1, sd1)
            wait_gather(gi0, gb0, ab0, sg0)

            @pl.when(i > 0)
            def _():
                wait_out(ob0, so0)

            reduce_write(r, gb0, ab0, ob0, so0)
            wait_dist(db0, sd0)
            select_issue(r + 2, db0, gi0, gb0, ab0, sg0)
            issue_dist(r + 4, db0, sd0)
            wait_gather(gi1, gb1, ab1, sg1)

            @pl.when(i > 0)
            def _():
                wait_out(ob1, so1)

            reduce_write(r + 1, gb1, ab1, ob1, so1)
            return c

        lax.fori_loop(0, RPW // 2, body, 0)
        wait_dist(db1, sd1)
        wait_dist(db0, sd0)
        wait_gather(gi0, gb0, ab0, sg0)
        wait_out(ob0, so0)
        wait_out(ob1, so1)

    if idxd is None:
        idxd = jnp.zeros((8,), jnp.int32)
    return sck(dist_flat, Ap, Bp, idxd)


def _layer2(xT, W, bnp, dbg_idx=False):
    B, N, Cp = xT.shape
    C = W.shape[1] // 2
    s, t = _bn_scale_shift(bnp)
    Wc = (W[:, :C] - W[:, C:]) * s[:, None]
    Wd = W[:, C:] * s[:, None]
    if C < Cp:
        Wc = jnp.pad(Wc, ((0, 0), (0, Cp - C)))
        Wd = jnp.pad(Wd, ((0, 0), (0, Cp - C)))
    dist, Ap, Bp = _layer_tc2(xT, Wc.T, Wd.T, t)
    O = Wc.shape[0]
    Bp2 = Bp.reshape(B * N, O)
    if O < 128:
        Bp2 = jnp.pad(Bp2, ((0, 0), (0, 128 - O)))
    idxd = None
    if dbg_idx:
        idx, _, _ = _layer_tc(xT, Wc.T, Wd.T, t)
        idxd = idx.reshape(-1)
    out = _sc_knn_max(dist.reshape(-1), Ap.reshape(B * N, O), Bp2, N, idxd=idxd)
    return out.reshape(B, N, O)


def _head_body(x1_ref, x2_ref, x3_ref, x4_ref, w1_ref, w2_ref, w3_ref, w4_ref,
               s5_ref, t5_ref, l1_ref, s6_ref, t6_ref,
               l2_ref, s7_ref, t7_ref, l3_ref, b3_ref, out_ref, *, n):
    dn = (((1,), (1,)), ((), ()))
    xe = (lax.dot_general(w1_ref[...], x1_ref[0], dn, preferred_element_type=jnp.float32)
          + lax.dot_general(w2_ref[...], x2_ref[0], dn, preferred_element_type=jnp.float32)
          + lax.dot_general(w3_ref[...], x3_ref[0], dn, preferred_element_type=jnp.float32)
          + lax.dot_general(w4_ref[...], x4_ref[0], dn, preferred_element_type=jnp.float32))
    xe = xe * s5_ref[...].T + t5_ref[...].T
    xe = jnp.maximum(xe, 0.2 * xe)
    xm = jnp.max(xe, axis=1)
    xa = jnp.sum(xe, axis=1) * (1.0 / n)
    xf = jnp.concatenate([xm, xa], axis=0)[None, :]
    h = jnp.dot(xf, l1_ref[...].T, preferred_element_type=jnp.float32) * s6_ref[...] + t6_ref[...]
    h = jnp.maximum(h, 0.2 * h)
    h = jnp.dot(h, l2_ref[...].T, preferred_element_type=jnp.float32) * s7_ref[...] + t7_ref[...]
    h = jnp.maximum(h, 0.2 * h)
    out_ref[0] = jnp.dot(h, l3_ref[...].T, preferred_element_type=jnp.float32) + b3_ref[...]


def _head(xs, params):
    B, N, _ = xs[0].shape
    s5, t5 = _bn_scale_shift(params['bn5'])
    s6, t6 = _bn_scale_shift(params['bn6'])
    s7, t7 = _bn_scale_shift(params['bn7'])
    W5 = params['W5']
    w5s = (W5[:, :64], W5[:, 64:128], W5[:, 128:256], W5[:, 256:512])
    in_specs = [pl.BlockSpec((1, N, xs[i].shape[2]), lambda b: (b, 0, 0)) for i in range(4)]
    in_specs += [pl.BlockSpec(w.shape, lambda b: tuple(0 for _ in w.shape)) for w in w5s]
    scalars = [s5[:, None], t5[:, None], params['L1'], s6[None, :], t6[None, :],
               params['L2'], s7[None, :], t7[None, :], params['L3'], params['L3b'][None, :]]
    in_specs += [pl.BlockSpec(a.shape, lambda b: tuple(0 for _ in a.shape)) for a in scalars]
    out = pl.pallas_call(
        functools.partial(_head_body, n=N),
        grid=(B,),
        in_specs=in_specs,
        out_specs=pl.BlockSpec((1, 1, 40), lambda b: (b, 0, 0)),
        out_shape=jax.ShapeDtypeStruct((B, 1, 40), jnp.float32),
    )(*xs, *w5s, *scalars)
    return out[:, 0, :]


def _layer(xT, W, bnp):
    B, N, Cp = xT.shape
    C2 = W.shape[1]
    C = C2 // 2
    s, t = _bn_scale_shift(bnp)
    Wc = (W[:, :C] - W[:, C:]) * s[:, None]
    Wd = W[:, C:] * s[:, None]
    if C < Cp:
        Wc = jnp.pad(Wc, ((0, 0), (0, Cp - C)))
        Wd = jnp.pad(Wd, ((0, 0), (0, Cp - C)))
    idx, Ap, Bp = _layer_tc(xT, Wc.T, Wd.T, t)
    O = Wc.shape[0]
    Bp2 = Bp.reshape(B * N, O)
    if O < 128:
        Bp2 = jnp.pad(Bp2, ((0, 0), (0, 128 - O)))
    out = _sc_gather_max(idx.reshape(-1), Ap.reshape(B * N, O), Bp2)
    return out.reshape(B, N, O)


def kernel(x, params):
    B, C0, N = x.shape
    xT = jnp.pad(jnp.swapaxes(x, 1, 2), ((0, 0), (0, 0), (0, 8 - C0)))
    x1 = _layer2(xT, params['W1'], params['bn1'])
    x2 = _layer2(x1, params['W2'], params['bn2'])
    x3 = _layer2(x2, params['W3'], params['bn3'])
    x4 = _layer2(x3, params['W4'], params['bn4'])
    return _head((x1, x2, x3, x4), params)


# speedup vs baseline: 1.6992x; 1.6992x over previous
import functools

import jax
import jax.numpy as jnp
from jax import lax
from jax.experimental import pallas as pl
from jax.experimental.pallas import tpu as pltpu
from jax.experimental.pallas import tpu_sc as plsc

KNN = 20
KPAD = 24


def _bn_scale_shift(p):
    s = p['g'] / jnp.sqrt(p['v'] + 1e-5)
    t = p['b'] - s * p['m']
    return s, t


def _prep_topk_body(x_ref, wc_ref, wd_ref, t_ref, idx_ref, a_ref, b_ref, *, n, k, kpad):
    b = pl.program_id(0)
    X = x_ref[0]
    a_ref[0] = jnp.dot(X, wc_ref[...], preferred_element_type=jnp.float32) + t_ref[...]
    b_ref[0] = jnp.dot(X, wd_ref[...], preferred_element_type=jnp.float32)
    G = lax.dot_general(X, X, (((1,), (1,)), ((), ())),
                        preferred_element_type=jnp.float32)
    xx = jnp.sum(X * X, axis=1, keepdims=True)
    work = xx + jnp.reshape(xx, (1, n)) - 2.0 * G
    iota = lax.broadcasted_iota(jnp.int32, (n, n), 1)
    base = b * n
    am = None
    for t in range(k):
        rowmin = jnp.min(work, axis=1, keepdims=True)
        cand = jnp.where(work == rowmin, iota, n)
        am = jnp.min(cand, axis=1)
        idx_ref[0, :, t] = am + base
        work = jnp.where(iota == am[:, None], jnp.inf, work)
    for t in range(k, kpad):
        idx_ref[0, :, t] = am + base


def _layer_tc(xT, wc_t, wd_t, tvec):
    B, N, Cp = xT.shape
    O = wc_t.shape[1]
    idx, Ap, Bp = pl.pallas_call(
        functools.partial(_prep_topk_body, n=N, k=KNN, kpad=KPAD),
        grid=(B,),
        in_specs=[
            pl.BlockSpec((1, N, Cp), lambda b: (b, 0, 0)),
            pl.BlockSpec((Cp, O), lambda b: (0, 0)),
            pl.BlockSpec((Cp, O), lambda b: (0, 0)),
            pl.BlockSpec((1, O), lambda b: (0, 0)),
        ],
        out_specs=[
            pl.BlockSpec((1, N, KPAD), lambda b: (b, 0, 0)),
            pl.BlockSpec((1, N, O), lambda b: (b, 0, 0)),
            pl.BlockSpec((1, N, O), lambda b: (b, 0, 0)),
        ],
        out_shape=[
            jax.ShapeDtypeStruct((B, N, KPAD), jnp.int32),
            jax.ShapeDtypeStruct((B, N, O), jnp.float32),
            jax.ShapeDtypeStruct((B, N, O), jnp.float32),
        ],
    )(xT, wc_t, wd_t, tvec[None, :])
    return idx, Ap, Bp


def _sc_gather_max(idx_flat, Ap, Bp):
    BN, O = Ap.shape
    WG = Bp.shape[1]
    K = KPAD
    CH = 4
    info = plsc.get_sparse_core_info()
    NC, NS = info.num_cores, info.num_subcores
    NW = NC * NS
    rows_per = BN // NW
    nch = rows_per // CH

    mesh = plsc.VectorSubcoreMesh(core_axis_name="c", subcore_axis_name="s")

    @functools.partial(
        pl.kernel, mesh=mesh,
        out_type=jax.ShapeDtypeStruct((BN, O), jnp.float32),
        scratch_types=[
            pltpu.VMEM((CH * K,), jnp.int32),
            pltpu.VMEM((CH * K,), jnp.int32),
            pltpu.VMEM((CH * K, WG), jnp.float32),
            pltpu.VMEM((CH * K, WG), jnp.float32),
            pltpu.VMEM((CH, O), jnp.float32),
            pltpu.VMEM((CH, O), jnp.float32),
            pltpu.VMEM((CH, O), jnp.float32),
            pltpu.SemaphoreType.DMA,
            pltpu.SemaphoreType.DMA,
        ],
    )
    def sck(idx_hbm, ap_hbm, bp_hbm, out_hbm, ib0, ib1, rb0, rb1, av0, av1, ob, s0, s1):
        wid = lax.axis_index("s") * NC + lax.axis_index("c")
        base = wid * rows_per

        def issue(ch, ib, rb, av, sem):
            r0 = base + ch * CH
            pltpu.sync_copy(idx_hbm.at[pl.ds(r0 * K, CH * K)], ib)
            pltpu.async_copy(bp_hbm.at[ib], rb, sem)
            pltpu.async_copy(ap_hbm.at[pl.ds(r0, CH)], av, sem)

        def wait(ib, rb, av, sem):
            pltpu.make_async_copy(bp_hbm.at[ib], rb, sem).wait()
            pltpu.make_async_copy(ap_hbm.at[pl.ds(0, CH)], av, sem).wait()

        def compute(ch, rb, av):
            r0 = base + ch * CH
            for rr in range(CH):
                for co in range(O // 16):
                    sl = pl.ds(co * 16, 16)

                    def jb(j, acc):
                        return jnp.maximum(acc, rb[rr * K + j, sl])

                    acc = lax.fori_loop(
                        0, K, jb, jnp.full((16,), -jnp.inf, jnp.float32))
                    y = av[rr, sl] + acc
                    ob[rr, sl] = jnp.maximum(y, 0.2 * y)
            pltpu.sync_copy(ob, out_hbm.at[pl.ds(r0, CH)])

        issue(0, ib0, rb0, av0, s0)

        def body(i, carry):
            ch = 2 * i
            issue(ch + 1, ib1, rb1, av1, s1)
            wait(ib0, rb0, av0, s0)
            compute(ch, rb0, av0)

            @pl.when(ch + 2 < nch)
            def _():
                issue(ch + 2, ib0, rb0, av0, s0)

            wait(ib1, rb1, av1, s1)
            compute(ch + 1, rb1, av1)
            return carry

        lax.fori_loop(0, nch // 2, body, 0)

    return sck(idx_flat, Ap, Bp)


def _prep_tau_body(x_ref, wc_ref, wd_ref, t_ref, dist_ref, a_ref, b_ref, *, n):
    X = x_ref[0]
    a_ref[0] = jnp.dot(X, wc_ref[...], preferred_element_type=jnp.float32) + t_ref[...]
    b_ref[0] = jnp.dot(X, wd_ref[...], preferred_element_type=jnp.float32)
    G = lax.dot_general(X, X, (((1,), (1,)), ((), ())),
                        preferred_element_type=jnp.float32)
    xx = jnp.sum(X * X, axis=1, keepdims=True)
    dist = xx + jnp.reshape(xx, (1, n)) - 2.0 * G
    dist_ref[0, :, :n] = dist
    w = dist
    for _ in range(5):
        h = w.shape[1] // 2
        w = jnp.minimum(w[:, :h], w[:, h:])
    io32 = lax.broadcasted_iota(jnp.int32, (n, 32), 1)
    for _ in range(12):
        rmax = jnp.max(w, axis=1, keepdims=True)
        cand = jnp.where(w == rmax, io32, -1)
        am = jnp.max(cand, axis=1, keepdims=True)
        w = jnp.where(io32 == am, -jnp.inf, w)
    tau = jnp.max(w, axis=1, keepdims=True)
    dist_ref[0, :, n:] = jnp.broadcast_to(tau, (n, 16))


def _layer_tc2(xT, wc_t, wd_t, tvec):
    B, N, Cp = xT.shape
    O = wc_t.shape[1]
    return pl.pallas_call(
        functools.partial(_prep_tau_body, n=N),
        grid=(B,),
        in_specs=[
            pl.BlockSpec((1, N, Cp), lambda b: (b, 0, 0)),
            pl.BlockSpec((Cp, O), lambda b: (0, 0)),
            pl.BlockSpec((Cp, O), lambda b: (0, 0)),
            pl.BlockSpec((1, O), lambda b: (0, 0)),
        ],
        out_specs=[
            pl.BlockSpec((1, N, N + 16), lambda b: (b, 0, 0)),
            pl.BlockSpec((1, N, O), lambda b: (b, 0, 0)),
            pl.BlockSpec((1, N, O), lambda b: (b, 0, 0)),
        ],
        out_shape=[
            jax.ShapeDtypeStruct((B, N, N + 16), jnp.float32),
            jax.ShapeDtypeStruct((B, N, O), jnp.float32),
            jax.ShapeDtypeStruct((B, N, O), jnp.float32),
        ],
    )(xT, wc_t, wd_t, tvec[None, :])


def _sc_knn_max(dist_flat, Ap, Bp, n, idxd=None):
    BN, O = Ap.shape
    WG = Bp.shape[1]
    use_idxd = idxd is not None
    K = KNN if not use_idxd else KPAD
    CV = n + 64
    info = plsc.get_sparse_core_info()
    NC, NS = info.num_cores, info.num_subcores
    NW = NC * NS
    RPW = BN // NW

    mesh = plsc.VectorSubcoreMesh(core_axis_name="c", subcore_axis_name="s")

    NR = n + 16

    @functools.partial(
        pl.kernel, mesh=mesh,
        out_type=jax.ShapeDtypeStruct((BN, O), jnp.float32),
        compiler_params=pltpu.CompilerParams(needs_layout_passes=False),
        scratch_types=[
            pltpu.VMEM((NR,), jnp.float32),
            pltpu.VMEM((NR,), jnp.float32),
            pltpu.VMEM((CV,), jnp.float32),
            pltpu.VMEM((CV,), jnp.int32),
            pltpu.VMEM((max(K, 8),), jnp.int32),
            pltpu.VMEM((max(K, 8),), jnp.int32),
            pltpu.VMEM((32 if not use_idxd else K, WG), jnp.float32),
            pltpu.VMEM((32 if not use_idxd else K, WG), jnp.float32),
            pltpu.VMEM((1, O), jnp.float32),
            pltpu.VMEM((1, O), jnp.float32),
            pltpu.VMEM((1, O), jnp.float32),
            pltpu.VMEM((1, O), jnp.float32),
            pltpu.SemaphoreType.DMA,
            pltpu.SemaphoreType.DMA,
            pltpu.SemaphoreType.DMA,
            pltpu.SemaphoreType.DMA,
            pltpu.SemaphoreType.DMA,
            pltpu.SemaphoreType.DMA,
        ],
    )
    def sck(dist_hbm, ap_hbm, bp_hbm, idxd_hbm, out_hbm,
            db0, db1, cv, cp, gi0, gi1, gb0, gb1, ab0, ab1, ob0, ob1,
            sd0, sd1, sg0, sg1, so0, so1):
        wid = lax.axis_index("s") * NC + lax.axis_index("c")
        base = wid * RPW
        sample_base = (base // n) * n
        iota16 = lax.iota(jnp.int32, 16)
        inf16 = jnp.full((16,), jnp.inf, jnp.float32)

        def m16(av, ap_, bv, bp_):
            rbv = lax.rev(bv, (0,))
            rbp = lax.rev(bp_, (0,))
            m = av <= rbv
            lv = jnp.where(m, av, rbv)
            lp = jnp.where(m, ap_, rbp)
            hv = jnp.where(m, rbv, av)
            hp = jnp.where(m, rbp, ap_)
            lv, lp = plsc.sort_key_val(lv, lp)
            hv, hp = plsc.sort_key_val(hv, hp)
            return lv, lp, hv, hp

        def merge_chunk(state, bv, bp_):
            slo_v, slo_p, shi_v, shi_p = state
            bv, bp_ = plsc.sort_key_val(bv, bp_)
            t0v, t0p, t1v, t1p = m16(shi_v, shi_p, bv, bp_)
            u0v, u0p, u1v, u1p = m16(slo_v, slo_p, t0v, t0p)
            v0v, v0p, _, _ = m16(u1v, u1p, t1v, t1p)
            return (u0v, u0p, v0v, v0p)

        def select_issue(r, db, gi, gb, ab_, sg):
            rc = jnp.minimum(r, RPW - 1)
            tau_s = db[pl.ds(n, 16)]
            cnt = jnp.zeros((16,), jnp.int32)
            for j in range(n // 16):
                v = db[pl.ds(16 * j, 16)]
                msk = v <= tau_s
                ones = jnp.where(msk, 1, 0)
                cs = plsc.cumsum(ones)
                pos = jnp.maximum(cnt + cs - 1, 0)
                plsc.store_scatter(cv, [pos], v, mask=msk)
                plsc.store_scatter(cp, [pos], iota16 + 16 * j, mask=msk)
                cnt = cnt + plsc.all_reduce_population_count(msk)
            for q in range(3):
                plsc.store_scatter(cv, [cnt + 16 * q + iota16], inf16)
            v0 = cv[pl.ds(0, 16)]
            slo_v, slo_p = plsc.sort_key_val(v0, cp[pl.ds(0, 16)])
            state = (slo_v, slo_p, inf16, iota16)
            for cidx in range(1, 4):
                bv = cv[pl.ds(16 * cidx, 16)]
                state = merge_chunk(state, bv, cp[pl.ds(16 * cidx, 16)])
            cnts = jnp.max(cnt)
            T = jnp.maximum(0, (cnts - 49) // 16)

            def tail(jj, st):
                off = 16 * (jj + 4)
                return merge_chunk(st, cv[pl.ds(off, 16)], cp[pl.ds(off, 16)])

            slo_v, slo_p, shi_v, shi_p = lax.fori_loop(0, T, tail, state)
            if use_idxd:
                pltpu.sync_copy(idxd_hbm.at[pl.ds((base + rc) * K, K)], gi)
                pltpu.async_copy(bp_hbm.at[gi], gb, sg)
            else:
                g0 = slo_p + sample_base
                g1m = jnp.where(iota16 < 4, shi_p, slo_p) + sample_base
                pltpu.async_copy(bp_hbm.at[g0], gb.at[pl.ds(0, 16)], sg)
                pltpu.async_copy(bp_hbm.at[g1m], gb.at[pl.ds(16, 16)], sg)
            pltpu.async_copy(ap_hbm.at[pl.ds(base + rc, 1)], ab_, sg)

        def wait_gather(gi, gb, ab_, sg):
            if use_idxd:
                pltpu.make_async_copy(bp_hbm.at[gi], gb, sg).wait()
            else:
                pltpu.make_async_copy(bp_hbm.at[iota16], gb.at[pl.ds(0, 16)], sg).wait()
                pltpu.make_async_copy(bp_hbm.at[iota16], gb.at[pl.ds(16, 16)], sg).wait()
            pltpu.make_async_copy(ap_hbm.at[pl.ds(0, 1)], ab_, sg).wait()

        def issue_dist(r, db, sd):
            rc = jnp.minimum(r, RPW - 1)
            pltpu.async_copy(dist_hbm.at[pl.ds((base + rc) * NR, NR)], db, sd)

        def wait_dist(db, sd):
            pltpu.make_async_copy(dist_hbm.at[pl.ds(0, NR)], db, sd).wait()

        def reduce_write(r, gb, ab_, ob_, so_):
            for co in range(O // 16):
                sl = pl.ds(co * 16, 16)
                acc = gb[0, sl]
                for j in range(1, K):
                    acc = jnp.maximum(acc, gb[j, sl])
                y = ab_[0, sl] + acc
                ob_[0, sl] = jnp.maximum(y, 0.2 * y)
            pltpu.async_copy(ob_, out_hbm.at[pl.ds(base + r, 1)], so_)

        def wait_out(ob_, so_):
            pltpu.make_async_copy(ob_, out_hbm.at[pl.ds(0, 1)], so_).wait()

        issue_dist(0, db0, sd0)
        issue_dist(1, db1, sd1)
        wait_dist(db0, sd0)
        select_issue(0, db0, gi0, gb0, ab0, sg0)
        issue_dist(2, db0, sd0)

        def body(i, c):
            r = 2 * i
            wait_dist(db1, sd1)
            select_issue(r + 1, db1, gi1, gb1, ab1, sg1)
            issue_dist(r + 3, db1, sd1)
            wait_gather(gi0, gb0, ab0, sg0)

            @pl.when(i > 0)
            def _():
                wait_out(ob0, so0)

            reduce_write(r, gb0, ab0, ob0, so0)
            wait_dist(db0, sd0)
            select_issue(r + 2, db0, gi0, gb0, ab0, sg0)
            issue_dist(r + 4, db0, sd0)
            wait_gather(gi1, gb1, ab1, sg1)

            @pl.when(i > 0)
            def _():
                wait_out(ob1, so1)

            reduce_write(r + 1, gb1, ab1, ob1, so1)
            return c

        lax.fori_loop(0, RPW // 2, body, 0)
        wait_dist(db1, sd1)
        wait_dist(db0, sd0)
        wait_gather(gi0, gb0, ab0, sg0)
        wait_out(ob0, so0)
        wait_out(ob1, so1)

    if idxd is None:
        idxd = jnp.zeros((8,), jnp.int32)
    return sck(dist_flat, Ap, Bp, idxd)


def _layer2(xT, W, bnp, dbg_idx=False):
    B, N, Cp = xT.shape
    C = W.shape[1] // 2
    s, t = _bn_scale_shift(bnp)
    Wc = (W[:, :C] - W[:, C:]) * s[:, None]
    Wd = W[:, C:] * s[:, None]
    if C < Cp:
        Wc = jnp.pad(Wc, ((0, 0), (0, Cp - C)))
        Wd = jnp.pad(Wd, ((0, 0), (0, Cp - C)))
    dist, Ap, Bp = _layer_tc2(xT, Wc.T, Wd.T, t)
    O = Wc.shape[0]
    Bp2 = Bp.reshape(B * N, O)
    if O < 128:
        Bp2 = jnp.pad(Bp2, ((0, 0), (0, 128 - O)))
    idxd = None
    if dbg_idx:
        idx, _, _ = _layer_tc(xT, Wc.T, Wd.T, t)
        idxd = idx.reshape(-1)
    out = _sc_knn_max(dist.reshape(-1), Ap.reshape(B * N, O), Bp2, N, idxd=idxd)
    return out.reshape(B, N, O)


def _sc_gather_max2(idx_flat, Ap, Bp):
    BN, O = Ap.shape
    WG = Bp.shape[1]
    K = KPAD
    info = plsc.get_sparse_core_info()
    NC, NS = info.num_cores, info.num_subcores
    NW = NC * NS
    RPW = BN // NW

    mesh = plsc.VectorSubcoreMesh(core_axis_name="c", subcore_axis_name="s")

    @functools.partial(
        pl.kernel, mesh=mesh,
        out_type=jax.ShapeDtypeStruct((BN, O), jnp.float32),
        compiler_params=pltpu.CompilerParams(needs_layout_passes=False),
        scratch_types=[
            pltpu.VMEM((RPW * K,), jnp.int32),
            pltpu.VMEM((K, WG), jnp.float32),
            pltpu.VMEM((K, WG), jnp.float32),
            pltpu.VMEM((1, O), jnp.float32),
            pltpu.VMEM((1, O), jnp.float32),
            pltpu.VMEM((1, O), jnp.float32),
            pltpu.VMEM((1, O), jnp.float32),
            pltpu.SemaphoreType.DMA,
            pltpu.SemaphoreType.DMA,
            pltpu.SemaphoreType.DMA,
            pltpu.SemaphoreType.DMA,
        ],
    )
    def sck(idx_hbm, ap_hbm, bp_hbm, out_hbm,
            idxv, gb0, gb1, ab0, ab1, ob0, ob1, sg0, sg1, so0, so1):
        wid = lax.axis_index("s") * NC + lax.axis_index("c")
        base = wid * RPW

        def issue(r, gb, ab_, sg):
            rc = jnp.minimum(r, RPW - 1)
            pltpu.async_copy(bp_hbm.at[idxv.at[pl.ds(rc * K, K)]], gb, sg)
            pltpu.async_copy(ap_hbm.at[pl.ds(base + rc, 1)], ab_, sg)

        def wait_gather(gb, ab_, sg):
            pltpu.make_async_copy(bp_hbm.at[idxv.at[pl.ds(0, K)]], gb, sg).wait()
            pltpu.make_async_copy(ap_hbm.at[pl.ds(0, 1)], ab_, sg).wait()

        def reduce_write(r, gb, ab_, ob_, so_):
            for co in range(O // 16):
                sl = pl.ds(co * 16, 16)
                acc = gb[0, sl]
                for j in range(1, K):
                    acc = jnp.maximum(acc, gb[j, sl])
                y = ab_[0, sl] + acc
                ob_[0, sl] = jnp.maximum(y, 0.2 * y)
            pltpu.async_copy(ob_, out_hbm.at[pl.ds(base + r, 1)], so_)

        def wait_out(ob_, so_):
            pltpu.make_async_copy(ob_, out_hbm.at[pl.ds(0, 1)], so_).wait()

        pltpu.sync_copy(idx_hbm.at[pl.ds(base * K, RPW * K)], idxv)
        issue(0, gb0, ab0, sg0)

        def body(i, c):
            r = 2 * i
            issue(r + 1, gb1, ab1, sg1)
            wait_gather(gb0, ab0, sg0)

            @pl.when(i > 0)
            def _():
                wait_out(ob0, so0)

            reduce_write(r, gb0, ab0, ob0, so0)
            issue(r + 2, gb0, ab0, sg0)
            wait_gather(gb1, ab1, sg1)

            @pl.when(i > 0)
            def _():
                wait_out(ob1, so1)

            reduce_write(r + 1, gb1, ab1, ob1, so1)
            return c

        lax.fori_loop(0, RPW // 2, body, 0)
        wait_gather(gb0, ab0, sg0)
        wait_out(ob0, so0)
        wait_out(ob1, so1)

    return sck(idx_flat, Ap, Bp)


def _layer25(xT, W, bnp):
    B, N, Cp = xT.shape
    C = W.shape[1] // 2
    s, t = _bn_scale_shift(bnp)
    Wc = (W[:, :C] - W[:, C:]) * s[:, None]
    Wd = W[:, C:] * s[:, None]
    if C < Cp:
        Wc = jnp.pad(Wc, ((0, 0), (0, Cp - C)))
        Wd = jnp.pad(Wd, ((0, 0), (0, Cp - C)))
    idx, Ap, Bp = _layer_tc(xT, Wc.T, Wd.T, t)
    O = Wc.shape[0]
    Bp2 = Bp.reshape(B * N, O)
    if O < 128:
        Bp2 = jnp.pad(Bp2, ((0, 0), (0, 128 - O)))
    out = _sc_gather_max2(idx.reshape(-1), Ap.reshape(B * N, O), Bp2)
    return out.reshape(B, N, O)


def _head_body(x1_ref, x2_ref, x3_ref, x4_ref, w1_ref, w2_ref, w3_ref, w4_ref,
               s5_ref, t5_ref, l1_ref, s6_ref, t6_ref,
               l2_ref, s7_ref, t7_ref, l3_ref, b3_ref, out_ref, *, n):
    dn = (((1,), (1,)), ((), ()))
    xe = (lax.dot_general(w1_ref[...], x1_ref[0], dn, preferred_element_type=jnp.float32)
          + lax.dot_general(w2_ref[...], x2_ref[0], dn, preferred_element_type=jnp.float32)
          + lax.dot_general(w3_ref[...], x3_ref[0], dn, preferred_element_type=jnp.float32)
          + lax.dot_general(w4_ref[...], x4_ref[0], dn, preferred_element_type=jnp.float32))
    xe = xe * s5_ref[...].T + t5_ref[...].T
    xe = jnp.maximum(xe, 0.2 * xe)
    xm = jnp.max(xe, axis=1)
    xa = jnp.sum(xe, axis=1) * (1.0 / n)
    xf = jnp.concatenate([xm, xa], axis=0)[None, :]
    h = jnp.dot(xf, l1_ref[...].T, preferred_element_type=jnp.float32) * s6_ref[...] + t6_ref[...]
    h = jnp.maximum(h, 0.2 * h)
    h = jnp.dot(h, l2_ref[...].T, preferred_element_type=jnp.float32) * s7_ref[...] + t7_ref[...]
    h = jnp.maximum(h, 0.2 * h)
    out_ref[0] = jnp.dot(h, l3_ref[...].T, preferred_element_type=jnp.float32) + b3_ref[...]


def _head(xs, params):
    B, N, _ = xs[0].shape
    s5, t5 = _bn_scale_shift(params['bn5'])
    s6, t6 = _bn_scale_shift(params['bn6'])
    s7, t7 = _bn_scale_shift(params['bn7'])
    W5 = params['W5']
    w5s = (W5[:, :64], W5[:, 64:128], W5[:, 128:256], W5[:, 256:512])
    in_specs = [pl.BlockSpec((1, N, xs[i].shape[2]), lambda b: (b, 0, 0)) for i in range(4)]
    in_specs += [pl.BlockSpec(w.shape, lambda b: tuple(0 for _ in w.shape)) for w in w5s]
    scalars = [s5[:, None], t5[:, None], params['L1'], s6[None, :], t6[None, :],
               params['L2'], s7[None, :], t7[None, :], params['L3'], params['L3b'][None, :]]
    in_specs += [pl.BlockSpec(a.shape, lambda b: tuple(0 for _ in a.shape)) for a in scalars]
    out = pl.pallas_call(
        functools.partial(_head_body, n=N),
        grid=(B,),
        in_specs=in_specs,
        out_specs=pl.BlockSpec((1, 1, 40), lambda b: (b, 0, 0)),
        out_shape=jax.ShapeDtypeStruct((B, 1, 40), jnp.float32),
    )(*xs, *w5s, *scalars)
    return out[:, 0, :]


def _layer(xT, W, bnp):
    B, N, Cp = xT.shape
    C2 = W.shape[1]
    C = C2 // 2
    s, t = _bn_scale_shift(bnp)
    Wc = (W[:, :C] - W[:, C:]) * s[:, None]
    Wd = W[:, C:] * s[:, None]
    if C < Cp:
        Wc = jnp.pad(Wc, ((0, 0), (0, Cp - C)))
        Wd = jnp.pad(Wd, ((0, 0), (0, Cp - C)))
    idx, Ap, Bp = _layer_tc(xT, Wc.T, Wd.T, t)
    O = Wc.shape[0]
    Bp2 = Bp.reshape(B * N, O)
    if O < 128:
        Bp2 = jnp.pad(Bp2, ((0, 0), (0, 128 - O)))
    out = _sc_gather_max(idx.reshape(-1), Ap.reshape(B * N, O), Bp2)
    return out.reshape(B, N, O)


def kernel(x, params):
    B, C0, N = x.shape
    xT = jnp.pad(jnp.swapaxes(x, 1, 2), ((0, 0), (0, 0), (0, 8 - C0)))
    x1 = _layer25(xT, params['W1'], params['bn1'])
    x2 = _layer25(x1, params['W2'], params['bn2'])
    x3 = _layer25(x2, params['W3'], params['bn3'])
    x4 = _layer25(x3, params['W4'], params['bn4'])
    return _head((x1, x2, x3, x4), params)
